# Initial kernel scaffold; baseline (speedup 1.0000x reference)
import jax
import jax.numpy as jnp
from jax.experimental import pallas as pl


def kernel(x, edge_index, edge_attr, enc_n_W1, enc_n_b1, enc_n_W2, enc_n_b2, enc_n_g, enc_n_be, enc_e_W1, enc_e_b1, enc_e_W2, enc_e_b2, enc_e_g, enc_e_be, le_W1, le_b1, le_W2, le_b2, le_g, le_be, ln_W1, ln_b1, ln_W2, ln_b2, ln_g, ln_be, dec_W1, dec_b1, dec_W2, dec_b2):
    raise NotImplementedError("write your pallas kernel here")


# speedup vs baseline: 1.8359x; 1.8359x over previous
import functools

import jax
import jax.numpy as jnp
from jax import lax
from jax.experimental import pallas as pl
from jax.experimental.pallas import tpu as pltpu
from jax.experimental.pallas import tpu_sc as plsc

N = 10000
NPAD = 10240
E = 160000
D = 256
ED = 16
H = 256
L = 5

NC = 2
NS = 16
NW = NC * NS


def _ln_rows(u, g, b):
    m = jnp.mean(u, axis=-1, keepdims=True)
    v = jnp.mean((u - m) ** 2, axis=-1, keepdims=True)
    return (u - m) * lax.rsqrt(v + 1e-5) * g + b


def _mlp2ln_body(x_ref, w1_ref, b1_ref, w2_ref, b2_ref, g_ref, be_ref, o_ref):
    hh = jnp.maximum(
        jnp.dot(x_ref[...], w1_ref[...], preferred_element_type=jnp.float32)
        + b1_ref[...], 0.0)
    u = jnp.dot(hh, w2_ref[...], preferred_element_type=jnp.float32) + b2_ref[...]
    o_ref[...] = _ln_rows(u, g_ref[...], be_ref[...])


def _mlp2ln(x, w1, b1, w2, b2, g, be, tile):
    r, din = x.shape
    hh = w1.shape[1]
    ho = w2.shape[1]
    return pl.pallas_call(
        _mlp2ln_body,
        grid=(r // tile,),
        in_specs=[
            pl.BlockSpec((tile, din), lambda i: (i, 0)),
            pl.BlockSpec((din, hh), lambda i: (0, 0)),
            pl.BlockSpec((1, hh), lambda i: (0, 0)),
            pl.BlockSpec((hh, ho), lambda i: (0, 0)),
            pl.BlockSpec((1, ho), lambda i: (0, 0)),
            pl.BlockSpec((1, ho), lambda i: (0, 0)),
            pl.BlockSpec((1, ho), lambda i: (0, 0)),
        ],
        out_specs=pl.BlockSpec((tile, ho), lambda i: (i, 0)),
        out_shape=jax.ShapeDtypeStruct((r, ho), jnp.float32),
    )(x, w1, b1.reshape(1, -1), w2, b2.reshape(1, -1), g.reshape(1, -1),
      be.reshape(1, -1))


def _pq_body(h_ref, ws_ref, wr_ref, p_ref, q_ref):
    hh = h_ref[...]
    p_ref[...] = jnp.dot(hh, ws_ref[...], preferred_element_type=jnp.float32)
    q_ref[...] = jnp.dot(hh, wr_ref[...], preferred_element_type=jnp.float32)


def _pq(h, ws, wr, tile=1024):
    return pl.pallas_call(
        _pq_body,
        grid=(NPAD // tile,),
        in_specs=[
            pl.BlockSpec((tile, D), lambda i: (i, 0)),
            pl.BlockSpec((D, H), lambda i: (0, 0)),
            pl.BlockSpec((D, H), lambda i: (0, 0)),
        ],
        out_specs=[
            pl.BlockSpec((tile, H), lambda i: (i, 0)),
            pl.BlockSpec((tile, H), lambda i: (i, 0)),
        ],
        out_shape=[jax.ShapeDtypeStruct((NPAD, H), jnp.float32)] * 2,
    )(h, ws, wr)


def _edge_body(gs_ref, gr_ref, e_ref, w1_ref, b1_ref, w2_ref, b2_ref, g_ref,
               be_ref, o_ref):
    e = e_ref[...]
    xin = (gs_ref[...] + gr_ref[...]
           + jnp.dot(e, w1_ref[...], preferred_element_type=jnp.float32)
           + b1_ref[...])
    hh = jnp.maximum(xin, 0.0)
    u = jnp.dot(hh, w2_ref[...], preferred_element_type=jnp.float32) + b2_ref[...]
    o_ref[...] = e + _ln_rows(u, g_ref[...], be_ref[...])


def _edge_layer(gs, gr, e, w1e, b1, w2, b2, g, be, tile=1600):
    return pl.pallas_call(
        _edge_body,
        grid=(E // tile,),
        in_specs=[
            pl.BlockSpec((tile, H), lambda i: (i, 0)),
            pl.BlockSpec((tile, H), lambda i: (i, 0)),
            pl.BlockSpec((tile, H), lambda i: (i, 0)),
            pl.BlockSpec((H, H), lambda i: (0, 0)),
            pl.BlockSpec((1, H), lambda i: (0, 0)),
            pl.BlockSpec((H, H), lambda i: (0, 0)),
            pl.BlockSpec((1, H), lambda i: (0, 0)),
            pl.BlockSpec((1, H), lambda i: (0, 0)),
            pl.BlockSpec((1, H), lambda i: (0, 0)),
        ],
        out_specs=pl.BlockSpec((tile, H), lambda i: (i, 0)),
        out_shape=jax.ShapeDtypeStruct((E, H), jnp.float32),
    )(gs, gr, e, w1e, b1.reshape(1, -1), w2, b2.reshape(1, -1),
      g.reshape(1, -1), be.reshape(1, -1))


def _node_body(h_ref, cs_ref, ce_ref, w1h_ref, w1a_ref, b1_ref, w2_ref,
               b2_ref, g_ref, be_ref, o_ref):
    hcur = h_ref[...]
    agg = ce_ref[...] - cs_ref[...]
    xin = (jnp.dot(hcur, w1h_ref[...], preferred_element_type=jnp.float32)
           + jnp.dot(agg, w1a_ref[...], preferred_element_type=jnp.float32)
           + b1_ref[...])
    hh = jnp.maximum(xin, 0.0)
    u = jnp.dot(hh, w2_ref[...], preferred_element_type=jnp.float32) + b2_ref[...]
    o_ref[...] = hcur + _ln_rows(u, g_ref[...], be_ref[...])


def _node_layer(h, cs, ce, w1h, w1a, b1, w2, b2, g, be, tile=1024):
    return pl.pallas_call(
        _node_body,
        grid=(NPAD // tile,),
        in_specs=[
            pl.BlockSpec((tile, D), lambda i: (i, 0)),
            pl.BlockSpec((tile, H), lambda i: (i, 0)),
            pl.BlockSpec((tile, H), lambda i: (i, 0)),
            pl.BlockSpec((D, H), lambda i: (0, 0)),
            pl.BlockSpec((H, H), lambda i: (0, 0)),
            pl.BlockSpec((1, H), lambda i: (0, 0)),
            pl.BlockSpec((H, D), lambda i: (0, 0)),
            pl.BlockSpec((1, D), lambda i: (0, 0)),
            pl.BlockSpec((1, D), lambda i: (0, 0)),
            pl.BlockSpec((1, D), lambda i: (0, 0)),
        ],
        out_specs=pl.BlockSpec((tile, D), lambda i: (i, 0)),
        out_shape=jax.ShapeDtypeStruct((NPAD, D), jnp.float32),
    )(h, cs, ce, w1h, w1a, b1.reshape(1, -1), w2, b2.reshape(1, -1),
      g.reshape(1, -1), be.reshape(1, -1))


CT = 256


def _cumsum_body(x_ref, l_ref, o_ref, carry_ref):
    i = pl.program_id(0)

    @pl.when(i == 0)
    def _():
        o_ref[...] = jnp.zeros_like(o_ref)
        carry_ref[...] = jnp.zeros_like(carry_ref)

    @pl.when(i > 0)
    def _():
        c = (jnp.dot(l_ref[...], x_ref[...], preferred_element_type=jnp.float32)
             + carry_ref[0:1, :])
        o_ref[...] = c
        carry_ref[0:1, :] = c[CT - 1:CT, :]


def _cumsum(e, ltri):
    return pl.pallas_call(
        _cumsum_body,
        grid=(E // CT + 1,),
        in_specs=[
            pl.BlockSpec((CT, H), lambda i: (jnp.where(i > 0, i - 1, 0), 0)),
            pl.BlockSpec((CT, CT), lambda i: (0, 0)),
        ],
        out_specs=pl.BlockSpec((CT, H), lambda i: (i, 0)),
        out_shape=jax.ShapeDtypeStruct((E + CT, H), jnp.float32),
        scratch_shapes=[pltpu.VMEM((8, H), jnp.float32)],
    )(e, ltri)


def _dec_body(h_ref, w1_ref, b1_ref, w2_ref, b2_ref, o_ref):
    hh = jnp.maximum(
        jnp.dot(h_ref[...], w1_ref[...], preferred_element_type=jnp.float32)
        + b1_ref[...], 0.0)
    o_ref[...] = (jnp.dot(hh, w2_ref[...], preferred_element_type=jnp.float32)
                  + b2_ref[...])


def _decoder(h, w1, b1, w2p, b2p, tile=1024):
    ho = w2p.shape[1]
    return pl.pallas_call(
        _dec_body,
        grid=(NPAD // tile,),
        in_specs=[
            pl.BlockSpec((tile, D), lambda i: (i, 0)),
            pl.BlockSpec((D, H), lambda i: (0, 0)),
            pl.BlockSpec((1, H), lambda i: (0, 0)),
            pl.BlockSpec((H, ho), lambda i: (0, 0)),
            pl.BlockSpec((1, ho), lambda i: (0, 0)),
        ],
        out_specs=pl.BlockSpec((tile, ho), lambda i: (i, 0)),
        out_shape=jax.ShapeDtypeStruct((NPAD, ho), jnp.float32),
    )(h, w1, b1.reshape(1, -1), w2p, b2p.reshape(1, -1))


_SC_MESH = plsc.VectorSubcoreMesh(
    core_axis_name="c", subcore_axis_name="s", num_cores=NC, num_subcores=NS)


def _make_dual_gather(length, ch, nbuf):
    nch = length // ch
    cpw = nch // NW
    extra = nch - NW * cpw
    assert length % ch == 0 and cpw % nbuf == 0 and extra <= NW
    idxlen = (cpw + (1 if extra else 0)) * ch

    def body(p_hbm, q_hbm, ia_hbm, ib_hbm, oa_hbm, ob_hbm,
             idx_a, idx_b, bufa, bufb, *sems):
        gsa = sems[0:nbuf]
        gsb = sems[nbuf:2 * nbuf]
        wsa = sems[2 * nbuf:3 * nbuf]
        wsb = sems[3 * nbuf:4 * nbuf]
        wid = lax.axis_index("s") * NC + lax.axis_index("c")
        ebase = wid * (cpw * ch)
        pltpu.sync_copy(ia_hbm.at[pl.ds(ebase, cpw * ch)],
                        idx_a.at[pl.ds(0, cpw * ch)])
        pltpu.sync_copy(ib_hbm.at[pl.ds(ebase, cpw * ch)],
                        idx_b.at[pl.ds(0, cpw * ch)])
        if extra:
            @pl.when(wid < extra)
            def _():
                tb = NW * cpw * ch + wid * ch
                pltpu.sync_copy(ia_hbm.at[pl.ds(tb, ch)],
                                idx_a.at[pl.ds(cpw * ch, ch)])
                pltpu.sync_copy(ib_hbm.at[pl.ds(tb, ch)],
                                idx_b.at[pl.ds(cpw * ch, ch)])

        def super_body(i, carry):
            base = i * nbuf
            descs = []
            for b in range(nbuf):
                @pl.when(i > 0)
                def _(b=b):
                    pltpu.make_async_copy(bufa.at[b], oa_hbm.at[pl.ds(0, ch)],
                                          wsa[b]).wait()
                    pltpu.make_async_copy(bufb.at[b], ob_hbm.at[pl.ds(0, ch)],
                                          wsb[b]).wait()
                off = (base + b) * ch
                da = pltpu.async_copy(p_hbm.at[idx_a.at[pl.ds(off, ch)]],
                                      bufa.at[b], gsa[b])
                db = pltpu.async_copy(q_hbm.at[idx_b.at[pl.ds(off, ch)]],
                                      bufb.at[b], gsb[b])
                descs.append((da, db))
            for b in range(nbuf):
                off = (base + b) * ch
                descs[b][0].wait()
                descs[b][1].wait()
                pltpu.async_copy(bufa.at[b], oa_hbm.at[pl.ds(ebase + off, ch)],
                                 wsa[b])
                pltpu.async_copy(bufb.at[b], ob_hbm.at[pl.ds(ebase + off, ch)],
                                 wsb[b])
            return carry

        lax.fori_loop(0, cpw // nbuf, super_body, 0)
        for b in range(nbuf):
            pltpu.make_async_copy(bufa.at[b], oa_hbm.at[pl.ds(0, ch)],
                                  wsa[b]).wait()
            pltpu.make_async_copy(bufb.at[b], ob_hbm.at[pl.ds(0, ch)],
                                  wsb[b]).wait()
        if extra:
            @pl.when(wid < extra)
            def _():
                off = cpw * ch
                gout = NW * cpw * ch + wid * ch
                da = pltpu.async_copy(p_hbm.at[idx_a.at[pl.ds(off, ch)]],
                                      bufa.at[0], gsa[0])
                db = pltpu.async_copy(q_hbm.at[idx_b.at[pl.ds(off, ch)]],
                                      bufb.at[0], gsb[0])
                da.wait()
                db.wait()
                wa = pltpu.async_copy(bufa.at[0], oa_hbm.at[pl.ds(gout, ch)],
                                      wsa[0])
                wb = pltpu.async_copy(bufb.at[0], ob_hbm.at[pl.ds(gout, ch)],
                                      wsb[0])
                wa.wait()
                wb.wait()

    return pl.kernel(
        body,
        out_type=[jax.ShapeDtypeStruct((length, H), jnp.float32)] * 2,
        mesh=_SC_MESH,
        scratch_types=[
            pltpu.VMEM((idxlen,), jnp.int32),
            pltpu.VMEM((idxlen,), jnp.int32),
            pltpu.VMEM((nbuf, ch, H), jnp.float32),
            pltpu.VMEM((nbuf, ch, H), jnp.float32),
        ] + [pltpu.SemaphoreType.DMA] * (4 * nbuf),
    )


_gather_call = _make_dual_gather(E, 64, 3)
_gatherN_call = _make_dual_gather(NPAD, 40, 4)


def kernel(x, edge_index, edge_attr,
           enc_n_W1, enc_n_b1, enc_n_W2, enc_n_b2, enc_n_g, enc_n_be,
           enc_e_W1, enc_e_b1, enc_e_W2, enc_e_b2, enc_e_g, enc_e_be,
           le_W1, le_b1, le_W2, le_b2, le_g, le_be,
           ln_W1, ln_b1, ln_W2, ln_b2, ln_g, ln_be,
           dec_W1, dec_b1, dec_W2, dec_b2):
    perm = jnp.argsort(edge_index[1])
    snd = edge_index[0][perm]
    rcv = edge_index[1][perm]
    ea = edge_attr[perm]
    cnt = jnp.cumsum(jnp.bincount(rcv, length=N), dtype=jnp.int32)
    gidx1 = jnp.pad(CT - 1 + cnt, (0, NPAD - N), constant_values=CT - 1)
    gidx0 = jnp.concatenate([jnp.full((1,), CT - 1, jnp.int32),
                             CT - 1 + cnt[:N - 1],
                             jnp.full((NPAD - N,), CT - 1, jnp.int32)])
    ltri = jnp.tril(jnp.ones((CT, CT), jnp.float32))

    xp = jnp.pad(x, ((0, NPAD - N), (0, 0)))
    h = _mlp2ln(xp, enc_n_W1, enc_n_b1, enc_n_W2, enc_n_b2, enc_n_g,
                enc_n_be, 1024)
    e = _mlp2ln(ea, enc_e_W1, enc_e_b1, enc_e_W2, enc_e_b2, enc_e_g,
                enc_e_be, 1600)
    for l in range(L):
        w1 = le_W1[l]
        p, q = _pq(h, w1[0:D], w1[D:2 * D])
        gs, gr = _gather_call(p, q, snd, rcv)
        e = _edge_layer(gs, gr, e, w1[2 * D:], le_b1[l], le_W2[l], le_b2[l],
                        le_g[l], le_be[l])
        c = _cumsum(e, ltri)
        cs, ce = _gatherN_call(c, c, gidx0, gidx1)
        h = _node_layer(h, cs, ce, ln_W1[l][:D], ln_W1[l][D:], ln_b1[l],
                        ln_W2[l], ln_b2[l], ln_g[l], ln_be[l])
    w2p = jnp.pad(dec_W2, ((0, 0), (0, 128 - dec_W2.shape[1])))
    b2p = jnp.pad(dec_b2, (0, 128 - dec_b2.shape[0]))
    out = _decoder(h, dec_W1, dec_b1, w2p, b2p)
    return out[:N, :3]


# speedup vs baseline: 2.3795x; 1.2961x over previous
import functools

import jax
import jax.numpy as jnp
from jax import lax
from jax.experimental import pallas as pl
from jax.experimental.pallas import tpu as pltpu
from jax.experimental.pallas import tpu_sc as plsc

N = 10000
NPAD = 10240
E = 160000
D = 256
ED = 16
H = 256
L = 5

NC = 2
NS = 16
NW = NC * NS


def _ln_rows(u, g, b):
    m = jnp.mean(u, axis=-1, keepdims=True)
    v = jnp.mean((u - m) ** 2, axis=-1, keepdims=True)
    return (u - m) * lax.rsqrt(v + 1e-5) * g + b


def _mlp2ln_body(x_ref, w1_ref, b1_ref, w2_ref, b2_ref, g_ref, be_ref, o_ref):
    hh = jnp.maximum(
        jnp.dot(x_ref[...], w1_ref[...], preferred_element_type=jnp.float32)
        + b1_ref[...], 0.0)
    u = jnp.dot(hh, w2_ref[...], preferred_element_type=jnp.float32) + b2_ref[...]
    o_ref[...] = _ln_rows(u, g_ref[...], be_ref[...])


def _mlp2ln(x, w1, b1, w2, b2, g, be, tile):
    r, din = x.shape
    hh = w1.shape[1]
    ho = w2.shape[1]
    return pl.pallas_call(
        _mlp2ln_body,
        grid=(r // tile,),
        in_specs=[
            pl.BlockSpec((tile, din), lambda i: (i, 0)),
            pl.BlockSpec((din, hh), lambda i: (0, 0)),
            pl.BlockSpec((1, hh), lambda i: (0, 0)),
            pl.BlockSpec((hh, ho), lambda i: (0, 0)),
            pl.BlockSpec((1, ho), lambda i: (0, 0)),
            pl.BlockSpec((1, ho), lambda i: (0, 0)),
            pl.BlockSpec((1, ho), lambda i: (0, 0)),
        ],
        out_specs=pl.BlockSpec((tile, ho), lambda i: (i, 0)),
        out_shape=jax.ShapeDtypeStruct((r, ho), jnp.float32),
    )(x, w1, b1.reshape(1, -1), w2, b2.reshape(1, -1), g.reshape(1, -1),
      be.reshape(1, -1))


def _pq_body(h_ref, ws_ref, wr_ref, p_ref, q_ref):
    hh = h_ref[...]
    p_ref[...] = jnp.dot(hh, ws_ref[...], preferred_element_type=jnp.float32)
    q_ref[...] = jnp.dot(hh, wr_ref[...], preferred_element_type=jnp.float32)


def _pq(h, ws, wr, tile=1024):
    return pl.pallas_call(
        _pq_body,
        grid=(NPAD // tile,),
        in_specs=[
            pl.BlockSpec((tile, D), lambda i: (i, 0)),
            pl.BlockSpec((D, H), lambda i: (0, 0)),
            pl.BlockSpec((D, H), lambda i: (0, 0)),
        ],
        out_specs=[
            pl.BlockSpec((tile, H), lambda i: (i, 0)),
            pl.BlockSpec((tile, H), lambda i: (i, 0)),
        ],
        out_shape=[jax.ShapeDtypeStruct((NPAD, H), jnp.float32)] * 2,
    )(h, ws, wr)


def _edge_body(gs_ref, gr_ref, e_ref, w1_ref, b1_ref, w2_ref, b2_ref, g_ref,
               be_ref, o_ref):
    e = e_ref[...]
    xin = (gs_ref[...] + gr_ref[...]
           + jnp.dot(e, w1_ref[...], preferred_element_type=jnp.float32)
           + b1_ref[...])
    hh = jnp.maximum(xin, 0.0)
    u = jnp.dot(hh, w2_ref[...], preferred_element_type=jnp.float32) + b2_ref[...]
    o_ref[...] = e + _ln_rows(u, g_ref[...], be_ref[...])


def _edge_layer(gs, gr, e, w1e, b1, w2, b2, g, be, tile=1600):
    return pl.pallas_call(
        _edge_body,
        grid=(E // tile,),
        in_specs=[
            pl.BlockSpec((tile, H), lambda i: (i, 0)),
            pl.BlockSpec((tile, H), lambda i: (i, 0)),
            pl.BlockSpec((tile, H), lambda i: (i, 0)),
            pl.BlockSpec((H, H), lambda i: (0, 0)),
            pl.BlockSpec((1, H), lambda i: (0, 0)),
            pl.BlockSpec((H, H), lambda i: (0, 0)),
            pl.BlockSpec((1, H), lambda i: (0, 0)),
            pl.BlockSpec((1, H), lambda i: (0, 0)),
            pl.BlockSpec((1, H), lambda i: (0, 0)),
        ],
        out_specs=pl.BlockSpec((tile, H), lambda i: (i, 0)),
        out_shape=jax.ShapeDtypeStruct((E, H), jnp.float32),
    )(gs, gr, e, w1e, b1.reshape(1, -1), w2, b2.reshape(1, -1),
      g.reshape(1, -1), be.reshape(1, -1))


def _node_body(h_ref, cs_ref, ce_ref, w1h_ref, w1a_ref, b1_ref, w2_ref,
               b2_ref, g_ref, be_ref, o_ref):
    hcur = h_ref[...]
    agg = ce_ref[...] - cs_ref[...]
    xin = (jnp.dot(hcur, w1h_ref[...], preferred_element_type=jnp.float32)
           + jnp.dot(agg, w1a_ref[...], preferred_element_type=jnp.float32)
           + b1_ref[...])
    hh = jnp.maximum(xin, 0.0)
    u = jnp.dot(hh, w2_ref[...], preferred_element_type=jnp.float32) + b2_ref[...]
    o_ref[...] = hcur + _ln_rows(u, g_ref[...], be_ref[...])


def _node_layer(h, cs, ce, w1h, w1a, b1, w2, b2, g, be, tile=1024):
    return pl.pallas_call(
        _node_body,
        grid=(NPAD // tile,),
        in_specs=[
            pl.BlockSpec((tile, D), lambda i: (i, 0)),
            pl.BlockSpec((tile, H), lambda i: (i, 0)),
            pl.BlockSpec((tile, H), lambda i: (i, 0)),
            pl.BlockSpec((D, H), lambda i: (0, 0)),
            pl.BlockSpec((H, H), lambda i: (0, 0)),
            pl.BlockSpec((1, H), lambda i: (0, 0)),
            pl.BlockSpec((H, D), lambda i: (0, 0)),
            pl.BlockSpec((1, D), lambda i: (0, 0)),
            pl.BlockSpec((1, D), lambda i: (0, 0)),
            pl.BlockSpec((1, D), lambda i: (0, 0)),
        ],
        out_specs=pl.BlockSpec((tile, D), lambda i: (i, 0)),
        out_shape=jax.ShapeDtypeStruct((NPAD, D), jnp.float32),
    )(h, cs, ce, w1h, w1a, b1.reshape(1, -1), w2, b2.reshape(1, -1),
      g.reshape(1, -1), be.reshape(1, -1))


CT = 256
CSUB = 5
CTILE = CT * CSUB


def _cumsum_body(x_ref, l_ref, o_ref, carry_ref):
    i = pl.program_id(0)

    @pl.when(i == 0)
    def _():
        o_ref[...] = jnp.zeros_like(o_ref)
        carry_ref[...] = jnp.zeros_like(carry_ref)

    @pl.when(i > 0)
    def _():
        lt = l_ref[...]
        for k in range(CSUB):
            c = (jnp.dot(lt, x_ref[k * CT:(k + 1) * CT, :],
                         preferred_element_type=jnp.float32)
                 + carry_ref[0:1, :])
            o_ref[k * CT:(k + 1) * CT, :] = c
            carry_ref[0:1, :] = c[CT - 1:CT, :]


def _cumsum(e, ltri):
    return pl.pallas_call(
        _cumsum_body,
        grid=(E // CTILE + 1,),
        in_specs=[
            pl.BlockSpec((CTILE, H), lambda i: (jnp.where(i > 0, i - 1, 0), 0)),
            pl.BlockSpec((CT, CT), lambda i: (0, 0)),
        ],
        out_specs=pl.BlockSpec((CTILE, H), lambda i: (i, 0)),
        out_shape=jax.ShapeDtypeStruct((E + CTILE, H), jnp.float32),
        scratch_shapes=[pltpu.VMEM((8, H), jnp.float32)],
    )(e, ltri)


def _dec_body(h_ref, w1_ref, b1_ref, w2_ref, b2_ref, o_ref):
    hh = jnp.maximum(
        jnp.dot(h_ref[...], w1_ref[...], preferred_element_type=jnp.float32)
        + b1_ref[...], 0.0)
    o_ref[...] = (jnp.dot(hh, w2_ref[...], preferred_element_type=jnp.float32)
                  + b2_ref[...])


def _decoder(h, w1, b1, w2p, b2p, tile=1024):
    ho = w2p.shape[1]
    return pl.pallas_call(
        _dec_body,
        grid=(NPAD // tile,),
        in_specs=[
            pl.BlockSpec((tile, D), lambda i: (i, 0)),
            pl.BlockSpec((D, H), lambda i: (0, 0)),
            pl.BlockSpec((1, H), lambda i: (0, 0)),
            pl.BlockSpec((H, ho), lambda i: (0, 0)),
            pl.BlockSpec((1, ho), lambda i: (0, 0)),
        ],
        out_specs=pl.BlockSpec((tile, ho), lambda i: (i, 0)),
        out_shape=jax.ShapeDtypeStruct((NPAD, ho), jnp.float32),
    )(h, w1, b1.reshape(1, -1), w2p, b2p.reshape(1, -1))


_SC_MESH = plsc.VectorSubcoreMesh(
    core_axis_name="c", subcore_axis_name="s", num_cores=NC, num_subcores=NS)


def _make_dual_gather(length, ch, nbuf):
    nch = length // ch
    cpw = nch // NW
    extra = nch - NW * cpw
    assert length % ch == 0 and cpw % nbuf == 0 and extra <= NW
    idxlen = (cpw + (1 if extra else 0)) * ch

    def body(p_hbm, q_hbm, ia_hbm, ib_hbm, oa_hbm, ob_hbm,
             idx_a, idx_b, bufa, bufb, *sems):
        gsa = sems[0:nbuf]
        gsb = sems[nbuf:2 * nbuf]
        wsa = sems[2 * nbuf:3 * nbuf]
        wsb = sems[3 * nbuf:4 * nbuf]
        wid = lax.axis_index("s") * NC + lax.axis_index("c")
        ebase = wid * (cpw * ch)
        pltpu.sync_copy(ia_hbm.at[pl.ds(ebase, cpw * ch)],
                        idx_a.at[pl.ds(0, cpw * ch)])
        pltpu.sync_copy(ib_hbm.at[pl.ds(ebase, cpw * ch)],
                        idx_b.at[pl.ds(0, cpw * ch)])
        if extra:
            @pl.when(wid < extra)
            def _():
                tb = NW * cpw * ch + wid * ch
                pltpu.sync_copy(ia_hbm.at[pl.ds(tb, ch)],
                                idx_a.at[pl.ds(cpw * ch, ch)])
                pltpu.sync_copy(ib_hbm.at[pl.ds(tb, ch)],
                                idx_b.at[pl.ds(cpw * ch, ch)])

        def super_body(i, carry):
            base = i * nbuf
            descs = []
            for b in range(nbuf):
                @pl.when(i > 0)
                def _(b=b):
                    pltpu.make_async_copy(bufa.at[b], oa_hbm.at[pl.ds(0, ch)],
                                          wsa[b]).wait()
                    pltpu.make_async_copy(bufb.at[b], ob_hbm.at[pl.ds(0, ch)],
                                          wsb[b]).wait()
                off = (base + b) * ch
                da = pltpu.async_copy(p_hbm.at[idx_a.at[pl.ds(off, ch)]],
                                      bufa.at[b], gsa[b])
                db = pltpu.async_copy(q_hbm.at[idx_b.at[pl.ds(off, ch)]],
                                      bufb.at[b], gsb[b])
                descs.append((da, db))
            for b in range(nbuf):
                off = (base + b) * ch
                descs[b][0].wait()
                descs[b][1].wait()
                pltpu.async_copy(bufa.at[b], oa_hbm.at[pl.ds(ebase + off, ch)],
                                 wsa[b])
                pltpu.async_copy(bufb.at[b], ob_hbm.at[pl.ds(ebase + off, ch)],
                                 wsb[b])
            return carry

        lax.fori_loop(0, cpw // nbuf, super_body, 0)
        for b in range(nbuf):
            pltpu.make_async_copy(bufa.at[b], oa_hbm.at[pl.ds(0, ch)],
                                  wsa[b]).wait()
            pltpu.make_async_copy(bufb.at[b], ob_hbm.at[pl.ds(0, ch)],
                                  wsb[b]).wait()
        if extra:
            @pl.when(wid < extra)
            def _():
                off = cpw * ch
                gout = NW * cpw * ch + wid * ch
                da = pltpu.async_copy(p_hbm.at[idx_a.at[pl.ds(off, ch)]],
                                      bufa.at[0], gsa[0])
                db = pltpu.async_copy(q_hbm.at[idx_b.at[pl.ds(off, ch)]],
                                      bufb.at[0], gsb[0])
                da.wait()
                db.wait()
                wa = pltpu.async_copy(bufa.at[0], oa_hbm.at[pl.ds(gout, ch)],
                                      wsa[0])
                wb = pltpu.async_copy(bufb.at[0], ob_hbm.at[pl.ds(gout, ch)],
                                      wsb[0])
                wa.wait()
                wb.wait()

    return pl.kernel(
        body,
        out_type=[jax.ShapeDtypeStruct((length, H), jnp.float32)] * 2,
        mesh=_SC_MESH,
        scratch_types=[
            pltpu.VMEM((idxlen,), jnp.int32),
            pltpu.VMEM((idxlen,), jnp.int32),
            pltpu.VMEM((nbuf, ch, H), jnp.float32),
            pltpu.VMEM((nbuf, ch, H), jnp.float32),
        ] + [pltpu.SemaphoreType.DMA] * (4 * nbuf),
    )


_gather_call = _make_dual_gather(E, 64, 3)
_gatherN_call = _make_dual_gather(NPAD, 40, 4)


def kernel(x, edge_index, edge_attr,
           enc_n_W1, enc_n_b1, enc_n_W2, enc_n_b2, enc_n_g, enc_n_be,
           enc_e_W1, enc_e_b1, enc_e_W2, enc_e_b2, enc_e_g, enc_e_be,
           le_W1, le_b1, le_W2, le_b2, le_g, le_be,
           ln_W1, ln_b1, ln_W2, ln_b2, ln_g, ln_be,
           dec_W1, dec_b1, dec_W2, dec_b2):
    perm = jnp.argsort(edge_index[1])
    snd = edge_index[0][perm]
    rcv = edge_index[1][perm]
    ea = edge_attr[perm]
    cnt = jnp.cumsum(jnp.bincount(rcv, length=N), dtype=jnp.int32)
    gidx1 = jnp.pad(CTILE - 1 + cnt, (0, NPAD - N), constant_values=CTILE - 1)
    gidx0 = jnp.concatenate([jnp.full((1,), CTILE - 1, jnp.int32),
                             CTILE - 1 + cnt[:N - 1],
                             jnp.full((NPAD - N,), CTILE - 1, jnp.int32)])
    ltri = jnp.tril(jnp.ones((CT, CT), jnp.float32))

    xp = jnp.pad(x, ((0, NPAD - N), (0, 0)))
    h = _mlp2ln(xp, enc_n_W1, enc_n_b1, enc_n_W2, enc_n_b2, enc_n_g,
                enc_n_be, 1024)
    e = _mlp2ln(ea, enc_e_W1, enc_e_b1, enc_e_W2, enc_e_b2, enc_e_g,
                enc_e_be, 1600)
    for l in range(L):
        w1 = le_W1[l]
        p, q = _pq(h, w1[0:D], w1[D:2 * D])
        gs, gr = _gather_call(p, q, snd, rcv)
        e = _edge_layer(gs, gr, e, w1[2 * D:], le_b1[l], le_W2[l], le_b2[l],
                        le_g[l], le_be[l])
        c = _cumsum(e, ltri)
        cs, ce = _gatherN_call(c, c, gidx0, gidx1)
        h = _node_layer(h, cs, ce, ln_W1[l][:D], ln_W1[l][D:], ln_b1[l],
                        ln_W2[l], ln_b2[l], ln_g[l], ln_be[l])
    w2p = jnp.pad(dec_W2, ((0, 0), (0, 128 - dec_W2.shape[1])))
    b2p = jnp.pad(dec_b2, (0, 128 - dec_b2.shape[0]))
    out = _decoder(h, dec_W1, dec_b1, w2p, b2p)
    return out[:N, :3]
